# baseline (speedup 1.0000x reference)
import functools

import jax
import jax.numpy as jnp
from jax import lax
from jax.experimental import pallas as pl
from jax.experimental.pallas import tpu as pltpu
from jax.experimental.pallas import tpu_sc as plsc

N = 10000
E = 320000
D = 128
A = 128

NC = 2
NS = 16
NW = NC * NS
EPW = E // NW
CH = 80
NCHUNK = EPW // CH
NZCH = N // CH

_mesh = plsc.VectorSubcoreMesh(core_axis_name="c", subcore_axis_name="s")


def _qk_body(x_ref, qwT_ref, kwT_ref, qb_ref, kb_ref, qx_ref, kx_ref):
    xv = x_ref[...]
    qx_ref[...] = (
        jnp.dot(xv, qwT_ref[...], preferred_element_type=jnp.float32)
        + qb_ref[...]
    )
    kx_ref[...] = (
        jnp.dot(xv, kwT_ref[...], preferred_element_type=jnp.float32)
        + kb_ref[...]
    )


def _qk_proj(x, qwT, kwT, qb, kb):
    return pl.pallas_call(
        _qk_body,
        out_shape=(
            jax.ShapeDtypeStruct((N, A), jnp.float32),
            jax.ShapeDtypeStruct((N, A), jnp.float32),
        ),
    )(x, qwT, kwT, qb, kb)


NBUF = 5


def _make_gather(ne, ch, nbuf):
    epw = ne // NW
    nchunk = epw // ch
    ngrp = nchunk // nbuf

    @functools.partial(
        pl.kernel,
        mesh=_mesh,
        out_type=jax.ShapeDtypeStruct((ne, A), jnp.float32),
        scratch_types=[
            pltpu.VMEM((nchunk, ch), jnp.int32),
            pltpu.VMEM((nchunk, ch), jnp.int32),
            pltpu.VMEM((nbuf, ch, A), jnp.float32),
        ] + [pltpu.SemaphoreType.DMA] * nbuf,
    )
    def gather(qx_hbm, kx_hbm, dst3_hbm, src3_hbm, s_hbm,
               didx_v, sidx_v, buf_v, *sems):
        wid = lax.axis_index("s") * NC + lax.axis_index("c")
        base0 = wid * epw
        pltpu.sync_copy(dst3_hbm.at[wid], didx_v)
        pltpu.sync_copy(src3_hbm.at[wid], sidx_v)

        def wait_slot(b):
            pltpu.make_async_copy(
                qx_hbm.at[didx_v.at[0]], buf_v.at[b], sems[b]
            ).wait()

        for b in range(nbuf):
            pltpu.async_copy(qx_hbm.at[didx_v.at[b]], buf_v.at[b], sems[b])

        def group(g, _):
            c0 = g * nbuf
            for b in range(nbuf):
                wait_slot(b)
                pltpu.async_copy(
                    kx_hbm.at[sidx_v.at[c0 + b]], buf_v.at[b], sems[b],
                    add=True,
                )
            for b in range(nbuf):
                wait_slot(b)
                pltpu.async_copy(
                    buf_v.at[b],
                    s_hbm.at[pl.ds(base0 + (c0 + b) * ch, ch)],
                    sems[b],
                )
            for b in range(nbuf):
                wait_slot(b)

                @pl.when(g < ngrp - 1)
                def _():
                    pltpu.async_copy(
                        qx_hbm.at[didx_v.at[c0 + nbuf + b]],
                        buf_v.at[b],
                        sems[b],
                    )
            return ()

        lax.fori_loop(0, ngrp, group, ())

    return gather


NSPLIT = 10
EH = E // NSPLIT
CHH = 40
NCHH = EH // NW // CHH
_gather_part = _make_gather(EH, CHH, NBUF)


RB = 2000
HB = EH // RB


def _edge_compute(s_ref, pc_ref, ewT_ref, cwT_ref, cb_ref, conn_ref):
    eh = jnp.dot(pc_ref[...], ewT_ref[...], preferred_element_type=jnp.float32)
    c1 = jnp.maximum(s_ref[...] + eh, 0.0)
    conn_ref[...] = (
        jnp.dot(c1, cwT_ref[...], preferred_element_type=jnp.float32)
        + cb_ref[...]
    )


def _edge_body0(s_ref, pc_ref, ewT_ref, cwT_ref, cb_ref, conn_ref):
    _edge_compute(s_ref, pc_ref, ewT_ref, cwT_ref, cb_ref, conn_ref)


def _edge_body1(s_ref, pc_ref, ewT_ref, cwT_ref, cb_ref, prev_ref, conn_ref):
    _edge_compute(s_ref, pc_ref, ewT_ref, cwT_ref, cb_ref, conn_ref)


def _make_edge_mlp(off, alias):
    in_specs = [
        pl.BlockSpec((RB, A), lambda i: (i, 0)),
        pl.BlockSpec((RB, D), lambda i, o=off: (i + o, 0)),
        pl.BlockSpec((D, A), lambda i: (0, 0)),
        pl.BlockSpec((A, D), lambda i: (0, 0)),
        pl.BlockSpec((1, D), lambda i: (0, 0)),
    ]
    kwargs = {}
    body = _edge_body0
    if alias:
        in_specs.append(pl.BlockSpec(memory_space=pltpu.HBM))
        body = _edge_body1
        kwargs["input_output_aliases"] = {5: 0}
    return pl.pallas_call(
        body,
        grid=(HB,),
        in_specs=in_specs,
        out_specs=pl.BlockSpec((RB, D), lambda i, o=off: (i + o, 0)),
        out_shape=jax.ShapeDtypeStruct((E, D), jnp.float32),
        compiler_params=pltpu.CompilerParams(
            dimension_semantics=("arbitrary",)
        ),
        **kwargs,
    )


CHD = 40
NCHD = EPW // CHD
NBUFD = 5
NGRPD = NCHD // NBUFD
NZCHD = N // CHD
NZK = (NZCHD + NS - 1) // NS


@functools.partial(
    pl.kernel,
    mesh=_mesh,
    out_type=jax.ShapeDtypeStruct((NC, N, D), jnp.float32),
    scratch_types=[
        pltpu.VMEM((NBUFD, CHD), jnp.int32),
        pltpu.VMEM((NBUFD, CHD, D), jnp.float32),
        pltpu.VMEM((CHD, D), jnp.float32),
        pltpu.VMEM_SHARED((N, D), jnp.float32),
    ] + [pltpu.SemaphoreType.DMA] * NBUFD,
)
def _scatter_agg(conn_hbm, dst_hbm, out_hbm,
                 idx_v, buf_v, zbuf_v, acc_sh, *sems):
    c = lax.axis_index("c")
    s = lax.axis_index("s")
    wid = s * NC + c
    base0 = wid * EPW

    zero = jnp.zeros((16,), jnp.float32)

    def zbody(r, _):
        for j in range(D // 16):
            zbuf_v[r, pl.ds(j * 16, 16)] = zero
        return ()

    lax.fori_loop(0, CHD, zbody, ())

    for k in range(NZK):
        chunk = s + k * NS

        @pl.when(chunk < NZCHD)
        def _():
            pltpu.sync_copy(zbuf_v, acc_sh.at[pl.ds(chunk * CHD, CHD)])

    plsc.subcore_barrier()

    def fire_loads(b, ch):
        base = base0 + ch * CHD
        pltpu.async_copy(dst_hbm.at[pl.ds(base, CHD)], idx_v.at[b], sems[b])
        pltpu.async_copy(conn_hbm.at[pl.ds(base, CHD)], buf_v.at[b], sems[b])

    def wait_idx(b):
        pltpu.make_async_copy(
            dst_hbm.at[pl.ds(0, CHD)], idx_v.at[b], sems[b]
        ).wait()

    def wait_rows(b):
        pltpu.make_async_copy(
            conn_hbm.at[pl.ds(0, CHD)], buf_v.at[b], sems[b]
        ).wait()

    for b in range(NBUFD):
        fire_loads(b, b)

    def group(g, _):
        c0 = g * NBUFD
        for b in range(NBUFD):
            wait_idx(b)
            wait_rows(b)
            pltpu.sync_copy(buf_v.at[b], acc_sh.at[idx_v.at[b]], add=True)

            @pl.when(g < NGRPD - 1)
            def _():
                fire_loads(b, c0 + NBUFD + b)
        return ()

    lax.fori_loop(0, NGRPD, group, ())
    plsc.subcore_barrier()

    for k in range(NZK):
        chunk = s + k * NS

        @pl.when(chunk < NZCHD)
        def _():
            pltpu.sync_copy(
                acc_sh.at[pl.ds(chunk * CHD, CHD)],
                out_hbm.at[c].at[pl.ds(chunk * CHD, CHD)],
            )


def _psum_body(p_ref, o_ref):
    o_ref[...] = p_ref[0] + p_ref[1]


def _psum(partial):
    return pl.pallas_call(
        _psum_body,
        out_shape=jax.ShapeDtypeStruct((N, D), jnp.float32),
    )(partial)


@jax.jit
def kernel(x, poly_conn, poly_index, Q_w, Q_b, K_w, K_b, E_w, conn_w, conn_b):
    dst = poly_index[0]
    src = poly_index[1]
    qx, kx = _qk_proj(x, Q_w.T, K_w.T, Q_b.reshape(1, A), K_b.reshape(1, A))
    ewT = E_w.T
    cwT = conn_w.T
    cb = conn_b.reshape(1, D)
    Ss = []
    for p in range(NSPLIT):
        dp = lax.slice(dst, (p * EH,), ((p + 1) * EH,)).reshape(NW, NCHH, CHH)
        sp = lax.slice(src, (p * EH,), ((p + 1) * EH,)).reshape(NW, NCHH, CHH)
        Ss.append(_gather_part(qx, kx, dp, sp))
    conn = None
    for p in range(NSPLIT):
        if conn is None:
            conn = _make_edge_mlp(0, False)(Ss[p], poly_conn, ewT, cwT, cb)
        else:
            conn = _make_edge_mlp(p * HB, True)(
                Ss[p], poly_conn, ewT, cwT, cb, conn
            )
    partial = _scatter_agg(conn, dst)
    agg = _psum(partial)
    return (agg, conn)


# speedup vs baseline: 1.0318x; 1.0318x over previous
import functools

import jax
import jax.numpy as jnp
from jax import lax
from jax.experimental import pallas as pl
from jax.experimental.pallas import tpu as pltpu
from jax.experimental.pallas import tpu_sc as plsc

N = 10000
E = 320000
D = 128
A = 128

NC = 2
NS = 16
NW = NC * NS
EPW = E // NW
CH = 80
NCHUNK = EPW // CH
NZCH = N // CH

_mesh = plsc.VectorSubcoreMesh(core_axis_name="c", subcore_axis_name="s")


def _qk_body(x_ref, qwT_ref, kwT_ref, qb_ref, kb_ref, qx_ref, kx_ref):
    xv = x_ref[...]
    qx_ref[...] = (
        jnp.dot(xv, qwT_ref[...], preferred_element_type=jnp.float32)
        + qb_ref[...]
    )
    kx_ref[...] = (
        jnp.dot(xv, kwT_ref[...], preferred_element_type=jnp.float32)
        + kb_ref[...]
    )


def _qk_proj(x, qwT, kwT, qb, kb):
    return pl.pallas_call(
        _qk_body,
        out_shape=(
            jax.ShapeDtypeStruct((N, A), jnp.float32),
            jax.ShapeDtypeStruct((N, A), jnp.float32),
        ),
    )(x, qwT, kwT, qb, kb)


NBUF = 5


def _make_gather(ne, ch, nbuf):
    epw = ne // NW
    nchunk = epw // ch
    ngrp = nchunk // nbuf

    @functools.partial(
        pl.kernel,
        mesh=_mesh,
        out_type=jax.ShapeDtypeStruct((ne, A), jnp.float32),
        scratch_types=[
            pltpu.VMEM((nchunk, ch), jnp.int32),
            pltpu.VMEM((nchunk, ch), jnp.int32),
            pltpu.VMEM((nbuf, ch, A), jnp.float32),
        ] + [pltpu.SemaphoreType.DMA] * (nbuf + 1),
    )
    def gather(qx_hbm, kx_hbm, dst3_hbm, src3_hbm, s_hbm,
               didx_v, sidx_v, buf_v, *sems):
        wid = lax.axis_index("s") * NC + lax.axis_index("c")
        base0 = wid * epw
        sidx_cp = pltpu.async_copy(src3_hbm.at[wid], sidx_v, sems[nbuf])
        pltpu.sync_copy(dst3_hbm.at[wid], didx_v)

        def wait_slot(b):
            pltpu.make_async_copy(
                qx_hbm.at[didx_v.at[0]], buf_v.at[b], sems[b]
            ).wait()

        for b in range(nbuf):
            pltpu.async_copy(qx_hbm.at[didx_v.at[b]], buf_v.at[b], sems[b])
        sidx_cp.wait()

        def group(g, _):
            c0 = g * nbuf
            for b in range(nbuf):
                wait_slot(b)
                pltpu.async_copy(
                    kx_hbm.at[sidx_v.at[c0 + b]], buf_v.at[b], sems[b],
                    add=True,
                )
            for b in range(nbuf):
                wait_slot(b)
                pltpu.async_copy(
                    buf_v.at[b],
                    s_hbm.at[pl.ds(base0 + (c0 + b) * ch, ch)],
                    sems[b],
                )
            for b in range(nbuf):
                wait_slot(b)

                @pl.when(g < ngrp - 1)
                def _():
                    pltpu.async_copy(
                        qx_hbm.at[didx_v.at[c0 + nbuf + b]],
                        buf_v.at[b],
                        sems[b],
                    )
            return ()

        lax.fori_loop(0, ngrp, group, ())

    return gather


NSPLIT = 5
EH = E // NSPLIT
CHH = 40
NCHH = EH // NW // CHH
_gather_part = _make_gather(EH, CHH, NBUF)


RB = 2000
HB = EH // RB


def _edge_compute(s_ref, pc_ref, ewT_ref, cwT_ref, cb_ref, conn_ref):
    eh = jnp.dot(pc_ref[...], ewT_ref[...], preferred_element_type=jnp.float32)
    c1 = jnp.maximum(s_ref[...] + eh, 0.0)
    conn_ref[...] = (
        jnp.dot(c1, cwT_ref[...], preferred_element_type=jnp.float32)
        + cb_ref[...]
    )


def _edge_body0(s_ref, pc_ref, ewT_ref, cwT_ref, cb_ref, conn_ref):
    _edge_compute(s_ref, pc_ref, ewT_ref, cwT_ref, cb_ref, conn_ref)


def _edge_body1(s_ref, pc_ref, ewT_ref, cwT_ref, cb_ref, prev_ref, conn_ref):
    _edge_compute(s_ref, pc_ref, ewT_ref, cwT_ref, cb_ref, conn_ref)


def _make_edge_mlp(off, alias):
    in_specs = [
        pl.BlockSpec((RB, A), lambda i: (i, 0)),
        pl.BlockSpec((RB, D), lambda i, o=off: (i + o, 0)),
        pl.BlockSpec((D, A), lambda i: (0, 0)),
        pl.BlockSpec((A, D), lambda i: (0, 0)),
        pl.BlockSpec((1, D), lambda i: (0, 0)),
    ]
    kwargs = {}
    body = _edge_body0
    if alias:
        in_specs.append(pl.BlockSpec(memory_space=pltpu.HBM))
        body = _edge_body1
        kwargs["input_output_aliases"] = {5: 0}
    return pl.pallas_call(
        body,
        grid=(HB,),
        in_specs=in_specs,
        out_specs=pl.BlockSpec((RB, D), lambda i, o=off: (i + o, 0)),
        out_shape=jax.ShapeDtypeStruct((E, D), jnp.float32),
        compiler_params=pltpu.CompilerParams(
            dimension_semantics=("arbitrary",)
        ),
        **kwargs,
    )


CHD = 40
NCHD = EPW // CHD
NBUFD = 5
NGRPD = NCHD // NBUFD
NZCHD = N // CHD
NZK = (NZCHD + NS - 1) // NS


@functools.partial(
    pl.kernel,
    mesh=_mesh,
    out_type=jax.ShapeDtypeStruct((NC, N, D), jnp.float32),
    scratch_types=[
        pltpu.VMEM((NBUFD, CHD), jnp.int32),
        pltpu.VMEM((NBUFD, CHD, D), jnp.float32),
        pltpu.VMEM((CHD, D), jnp.float32),
        pltpu.VMEM_SHARED((N, D), jnp.float32),
    ] + [pltpu.SemaphoreType.DMA] * NBUFD,
)
def _scatter_agg(conn_hbm, dst_hbm, out_hbm,
                 idx_v, buf_v, zbuf_v, acc_sh, *sems):
    c = lax.axis_index("c")
    s = lax.axis_index("s")
    wid = s * NC + c
    base0 = wid * EPW

    zero = jnp.zeros((16,), jnp.float32)

    def zbody(r, _):
        for j in range(D // 16):
            zbuf_v[r, pl.ds(j * 16, 16)] = zero
        return ()

    lax.fori_loop(0, CHD, zbody, ())

    for k in range(NZK):
        chunk = s + k * NS

        @pl.when(chunk < NZCHD)
        def _():
            pltpu.sync_copy(zbuf_v, acc_sh.at[pl.ds(chunk * CHD, CHD)])

    plsc.subcore_barrier()

    def fire_loads(b, ch):
        base = base0 + ch * CHD
        pltpu.async_copy(dst_hbm.at[pl.ds(base, CHD)], idx_v.at[b], sems[b])
        pltpu.async_copy(conn_hbm.at[pl.ds(base, CHD)], buf_v.at[b], sems[b])

    def wait_idx(b):
        pltpu.make_async_copy(
            dst_hbm.at[pl.ds(0, CHD)], idx_v.at[b], sems[b]
        ).wait()

    def wait_rows(b):
        pltpu.make_async_copy(
            conn_hbm.at[pl.ds(0, CHD)], buf_v.at[b], sems[b]
        ).wait()

    for b in range(NBUFD):
        fire_loads(b, b)

    def group(g, _):
        c0 = g * NBUFD
        for b in range(NBUFD):
            wait_idx(b)
            wait_rows(b)
            pltpu.sync_copy(buf_v.at[b], acc_sh.at[idx_v.at[b]], add=True)

            @pl.when(g < NGRPD - 1)
            def _():
                fire_loads(b, c0 + NBUFD + b)
        return ()

    lax.fori_loop(0, NGRPD, group, ())
    plsc.subcore_barrier()

    for k in range(NZK):
        chunk = s + k * NS

        @pl.when(chunk < NZCHD)
        def _():
            pltpu.sync_copy(
                acc_sh.at[pl.ds(chunk * CHD, CHD)],
                out_hbm.at[c].at[pl.ds(chunk * CHD, CHD)],
            )


def _psum_body(p_ref, o_ref):
    o_ref[...] = p_ref[0] + p_ref[1]


def _psum(partial):
    return pl.pallas_call(
        _psum_body,
        out_shape=jax.ShapeDtypeStruct((N, D), jnp.float32),
    )(partial)


@jax.jit
def kernel(x, poly_conn, poly_index, Q_w, Q_b, K_w, K_b, E_w, conn_w, conn_b):
    dst = poly_index[0]
    src = poly_index[1]
    qx, kx = _qk_proj(x, Q_w.T, K_w.T, Q_b.reshape(1, A), K_b.reshape(1, A))
    ewT = E_w.T
    cwT = conn_w.T
    cb = conn_b.reshape(1, D)
    Ss = []
    for p in range(NSPLIT):
        dp = lax.slice(dst, (p * EH,), ((p + 1) * EH,)).reshape(NW, NCHH, CHH)
        sp = lax.slice(src, (p * EH,), ((p + 1) * EH,)).reshape(NW, NCHH, CHH)
        Ss.append(_gather_part(qx, kx, dp, sp))
    conn = None
    for p in range(NSPLIT):
        if conn is None:
            conn = _make_edge_mlp(0, False)(Ss[p], poly_conn, ewT, cwT, cb)
        else:
            conn = _make_edge_mlp(p * HB, True)(
                Ss[p], poly_conn, ewT, cwT, cb, conn
            )
    partial = _scatter_agg(conn, dst)
    agg = _psum(partial)
    return (agg, conn)
